# baseline (speedup 1.0000x reference)
import functools

import jax
import jax.numpy as jnp
from jax import lax
from jax.experimental import pallas as pl
from jax.experimental.pallas import tpu as pltpu
from jax.experimental.pallas import tpu_sc as plsc

_SUP = (12.0, 53.0, 31.0, 23.0)
_INF = (1.0, 1.0, 1.0, 0.0)
_NC, _NS, _L = 2, 16, 16
_NW = _NC * _NS
_JB = 4
_IB = 32
_NB = 4


@functools.lru_cache(maxsize=None)
def _build(n_i, n_j):
    slab = n_i // _NW
    n_jc = n_j // _JB
    n_sub = slab // _IB
    assert slab * _NW == n_i and n_jc * _JB == n_j and n_sub * _IB == slab
    assert n_sub % _NB == 0 and slab % 128 == 0

    def body(ts_hbm, tbl_hbm, out_hbm, ts_v, out_v, tbl_v,
             in_sem0, in_sem1, out_sem0, out_sem1, out_sem2, out_sem3):
        in_sems = (in_sem0, in_sem1)
        out_sems = (out_sem0, out_sem1, out_sem2, out_sem3)
        wid = lax.axis_index("s") * _NC + lax.axis_index("c")
        i0 = wid * slab

        pltpu.sync_copy(tbl_hbm, tbl_v)
        ab = []
        for f in range(4):
            inv = 1.0 / (_INF[f] - _SUP[f])
            for h in range(2):
                e0 = tbl_v[pl.ds(64 * f + 16 * h, _L)]
                e1 = tbl_v[pl.ds(64 * f + 32 + 16 * h, _L)]
                ab.append(((e1 - e0) * inv,
                           (_SUP[f] * e0 - _INF[f] * e1) * inv))

        def in_copy(jc, b, d):
            return pltpu.make_async_copy(
                ts_hbm.at[d + 1, pl.ds(jc * _JB, _JB), pl.ds(i0, slab)],
                ts_v.at[b, d], in_sems[b])

        def out_copy(jc, k, b2):
            return pltpu.make_async_copy(
                out_v.at[b2],
                out_hbm.at[pl.ds(i0 + k * _IB, _IB),
                           pl.ds(jc * _JB, _JB), pl.ds(0, 128)],
                out_sems[b2])

        def compute(jc, k, b, b2):
            @plsc.parallel_loop(0, _JB * (_IB // 16), unroll=1)
            def group(gi):
                jj = gi >> 1
                g = gi & 1
                vecs = [ts_v[b, d, jj, pl.ds(k * _IB + g * 16, _L)]
                        for d in range(4)]
                for ii in range(16):
                    row = g * 16 + ii
                    for f in range(4):
                        v = vecs[f][ii]
                        for h in range(2):
                            a, c = ab[2 * f + h]
                            out_v[b2, row, jj, pl.ds(32 * f + 16 * h, _L)] = (
                                v * a + c)

        def do_chunk(jc, b):
            for d in range(4):
                in_copy(jc, b, d).wait()

            @pl.when(jc + 1 < n_jc)
            def _next_in():
                for d in range(4):
                    in_copy(jc + 1, 1 - b, d).start()

            def sub(k2, carry2):
                for b2 in range(_NB):
                    k = k2 * _NB + b2
                    s = jc * n_sub + k

                    @pl.when(s >= _NB)
                    def _wait_out():
                        out_copy(jc, k, b2).wait()

                    compute(jc, k, b, b2)
                    out_copy(jc, k, b2).start()
                return carry2
            lax.fori_loop(0, n_sub // _NB, sub, 0)

        for d in range(4):
            in_copy(0, 0, d).start()

        def jchunk(jc2, carry):
            for b in range(2):
                do_chunk(jc2 * 2 + b, b)
            return carry
        lax.fori_loop(0, n_jc // 2, jchunk, 0)
        if n_jc % 2:
            do_chunk(jnp.int32(n_jc - 1), (n_jc - 1) % 2)

        for b2 in range(_NB):
            out_copy(n_jc - 1, n_sub - _NB + b2, b2).wait()

    return pl.kernel(
        body,
        out_type=jax.ShapeDtypeStruct((n_i, n_j, 128), jnp.float32),
        mesh=plsc.VectorSubcoreMesh(
            core_axis_name="c", subcore_axis_name="s",
            num_cores=_NC, num_subcores=_NS),
        scratch_types=[
            pltpu.VMEM((2, 4, _JB, slab), jnp.float32),
            pltpu.VMEM((_NB, _IB, _JB, 128), jnp.float32),
            pltpu.VMEM((4 * 64,), jnp.float32),
            pltpu.SemaphoreType.DMA,
            pltpu.SemaphoreType.DMA,
            pltpu.SemaphoreType.DMA,
            pltpu.SemaphoreType.DMA,
            pltpu.SemaphoreType.DMA,
            pltpu.SemaphoreType.DMA,
        ],
    )


def kernel(timestamp, M_w, W_w, D_w, H_w):
    shape = timestamp.shape[:-1]
    n_rows = 1
    for d in shape:
        n_rows *= d
    n_j = timestamp.shape[-2]
    n_i = n_rows // n_j
    tsT = jnp.transpose(timestamp.reshape(n_i, n_j, 5), (2, 1, 0))
    tbl = jnp.concatenate([M_w.reshape(-1), W_w.reshape(-1),
                           D_w.reshape(-1), H_w.reshape(-1)])
    out = _build(n_i, n_j)(tsT, tbl)
    return out.reshape(*shape, 128)


# speedup vs baseline: 1.3013x; 1.3013x over previous
import functools

import jax
import jax.numpy as jnp
from jax import lax
from jax.experimental import pallas as pl
from jax.experimental.pallas import tpu as pltpu
from jax.experimental.pallas import tpu_sc as plsc

_SUP = (12.0, 53.0, 31.0, 23.0)
_INF = (1.0, 1.0, 1.0, 0.0)
_NC, _NS, _L = 2, 16, 16
_NW = _NC * _NS
_JB = 4
_IB = 32
_NB = 2


@functools.lru_cache(maxsize=None)
def _build(n_i, n_j):
    slab = n_i // _NW
    n_jc = n_j // _JB
    n_sub = slab // _IB
    assert slab * _NW == n_i and n_jc * _JB == n_j and n_sub * _IB == slab
    assert n_sub % _NB == 0 and slab % 128 == 0

    def body(ts_hbm, tbl_hbm, out_hbm, ts_v, out_v, tbl_v,
             in_sem0, in_sem1, out_sem0, out_sem1):
        in_sems = (in_sem0, in_sem1)
        out_sems = (out_sem0, out_sem1)
        wid = lax.axis_index("s") * _NC + lax.axis_index("c")
        i0 = wid * slab

        pltpu.sync_copy(tbl_hbm, tbl_v)
        ab = []
        for f in range(4):
            inv = 1.0 / (_INF[f] - _SUP[f])
            for h in range(2):
                e0 = tbl_v[pl.ds(64 * f + 16 * h, _L)]
                e1 = tbl_v[pl.ds(64 * f + 32 + 16 * h, _L)]
                ab.append(((e1 - e0) * inv,
                           (_SUP[f] * e0 - _INF[f] * e1) * inv))

        def in_copy(jc, b, d):
            return pltpu.make_async_copy(
                ts_hbm.at[d + 1, pl.ds(jc * _JB, _JB), pl.ds(i0, slab)],
                ts_v.at[b, d], in_sems[b])

        def out_copy(jc, k, b2):
            return pltpu.make_async_copy(
                out_v.at[b2],
                out_hbm.at[pl.ds(i0 + k * _IB, _IB),
                           pl.ds(jc * _JB, _JB), pl.ds(0, 128)],
                out_sems[b2])

        def compute(jc, k, b, b2):
            @plsc.parallel_loop(0, _JB * (_IB // 16), unroll=1)
            def group(gi):
                jj = gi >> 1
                g = gi & 1
                vecs = [ts_v[b, d, jj, pl.ds(k * _IB + g * 16, _L)]
                        for d in range(4)]
                for ii in range(16):
                    row = g * 16 + ii
                    for f in range(4):
                        v = vecs[f][ii]
                        for h in range(2):
                            a, c = ab[2 * f + h]
                            out_v[b2, row, jj, pl.ds(32 * f + 16 * h, _L)] = (
                                v * a + c)

        def do_chunk(jc, b):
            for d in range(4):
                in_copy(jc, b, d).wait()

            @pl.when(jc + 1 < n_jc)
            def _next_in():
                for d in range(4):
                    in_copy(jc + 1, 1 - b, d).start()

            def sub(k2, carry2):
                for b2 in range(_NB):
                    k = k2 * _NB + b2
                    s = jc * n_sub + k

                    @pl.when(s >= _NB)
                    def _wait_out():
                        out_copy(jc, k, b2).wait()

                    compute(jc, k, b, b2)
                    out_copy(jc, k, b2).start()
                return carry2
            lax.fori_loop(0, n_sub // _NB, sub, 0)

        for d in range(4):
            in_copy(0, 0, d).start()

        def jchunk(jc2, carry):
            for b in range(2):
                do_chunk(jc2 * 2 + b, b)
            return carry
        lax.fori_loop(0, n_jc // 2, jchunk, 0)
        if n_jc % 2:
            do_chunk(jnp.int32(n_jc - 1), (n_jc - 1) % 2)

        for b2 in range(_NB):
            out_copy(n_jc - 1, n_sub - _NB + b2, b2).wait()

    return pl.kernel(
        body,
        out_type=jax.ShapeDtypeStruct((n_i, n_j, 128), jnp.float32),
        mesh=plsc.VectorSubcoreMesh(
            core_axis_name="c", subcore_axis_name="s",
            num_cores=_NC, num_subcores=_NS),
        scratch_types=[
            pltpu.VMEM((2, 4, _JB, slab), jnp.float32),
            pltpu.VMEM((_NB, _IB, _JB, 128), jnp.float32),
            pltpu.VMEM((4 * 64,), jnp.float32),
            pltpu.SemaphoreType.DMA,
            pltpu.SemaphoreType.DMA,
            pltpu.SemaphoreType.DMA,
            pltpu.SemaphoreType.DMA,
        ],
    )


def kernel(timestamp, M_w, W_w, D_w, H_w):
    shape = timestamp.shape[:-1]
    n_rows = 1
    for d in shape:
        n_rows *= d
    n_j = timestamp.shape[-2]
    n_i = n_rows // n_j
    tsT = jnp.transpose(timestamp.reshape(n_i, n_j, 5), (2, 1, 0))
    tbl = jnp.concatenate([M_w.reshape(-1), W_w.reshape(-1),
                           D_w.reshape(-1), H_w.reshape(-1)])
    out = _build(n_i, n_j)(tsT, tbl)
    return out.reshape(*shape, 128)
